# baseline (speedup 1.0000x reference)
import functools

import jax
import jax.numpy as jnp
from jax import lax
from jax.experimental import pallas as pl
from jax.experimental.pallas import tpu as pltpu
from jax.experimental.pallas import tpu_sc as plsc

R, N = 128, 32768
K = 256
NC, NS, L = 2, 16, 16
NW = NC * NS
ROWS_PER_W = R // NW
CHUNKS = N // L
T0 = 2.0
T0KEY = 0x40000000
RBITS = 19
RBUCKETS = 256
CAP = 32000
IMIN = -2147483648


def _scalar(x):
    return x if x.ndim == 0 else x[0]


def _keys(v):
    b = lax.bitcast_convert_type(v, jnp.int32)
    return jnp.where(b >= 0, b, b ^ jnp.int32(0x7FFFFFFF))


def _body(x_hbm, out_hbm, rowa_v, rowb_v, hist_v, canda_v, candb_v,
          outrow_v, sema, semb):
    wid = lax.axis_index("s") * NC + lax.axis_index("c")
    iota16 = lax.iota(jnp.int32, L)
    ones = jnp.ones((L,), jnp.int32)
    zeros16 = jnp.zeros((L,), jnp.int32)

    def count_ge_ref(ref, tt, ncc, nvalid):
        def cnt(cc, acc):
            key = _keys(ref[pl.ds(cc * L, L)])
            valid = (cc * L + iota16) < nvalid
            ge = jnp.logical_and(key >= tt, valid)
            return acc + jnp.where(ge, ones, zeros16)

        return jnp.sum(lax.fori_loop(0, ncc, cnt, zeros16))

    def bit_search(ref, t0, nbits, ncc, nvalid, ktarget):
        def bit_body(i, t):
            tt = t | (jnp.int32(1) << (nbits - 1 - i))
            c_ge = count_ge_ref(ref, tt, ncc, nvalid)
            return jnp.where(c_ge >= ktarget, tt, t)

        return lax.fori_loop(0, nbits, bit_body, t0)

    def emit_selection(ref, nvalid, ncc, tkey, c_gt, c_ge):
        def simple_sel(_):
            def sp(cc, off):
                v = ref[pl.ds(cc * L, L)]
                key = _keys(v)
                valid = (cc * L + iota16) < nvalid
                m = jnp.logical_and(key >= tkey, valid)
                plsc.store_compressed(
                    outrow_v.at[pl.ds(off, L)], v, mask=m)
                return off + _scalar(plsc.all_reduce_population_count(m))

            lax.fori_loop(0, ncc, sp, jnp.int32(0))
            return 0

        def tie_sel(_):
            needed_eq = K - c_gt

            def sp(cc, st):
                off, cnt_eq = st
                v = ref[pl.ds(cc * L, L)]
                key = _keys(v)
                valid = (cc * L + iota16) < nvalid
                meq = jnp.logical_and(key == tkey, valid)
                cum = plsc.cumsum(meq.astype(jnp.int32))
                sel_eq = jnp.logical_and(
                    meq, (cnt_eq + cum) <= needed_eq)
                m = jnp.logical_or(
                    jnp.logical_and(key > tkey, valid), sel_eq)
                plsc.store_compressed(
                    outrow_v.at[pl.ds(off, L)], v, mask=m)
                return (off + _scalar(
                            plsc.all_reduce_population_count(m)),
                        cnt_eq + cum[15])

            lax.fori_loop(0, ncc, sp, (jnp.int32(0), jnp.int32(0)))
            return 0

        lax.cond(c_ge == K, simple_sel, tie_sel, 0)

    def do_tail(row, buf_v, cand_v, nc0, ovf):
        def path_fast(_):
            ncc = (nc0 + (L - 1)) // L

            def zero_hist(i, _c):
                hist_v[pl.ds(i * L, L)] = zeros16
                return 0

            lax.fori_loop(0, RBUCKETS // L, zero_hist, 0)

            def hist_pass(cc, _c):
                key = _keys(cand_v[pl.ds(cc * L, L)])
                valid = (cc * L + iota16) < nc0
                bkt = jnp.minimum(
                    (key - jnp.int32(T0KEY)) >> RBITS,
                    jnp.int32(RBUCKETS - 1))
                plsc.addupdate_scatter(
                    hist_v, [bkt], jnp.where(valid, ones, zeros16))
                return 0

            lax.fori_loop(0, ncc, hist_pass, 0)

            def scan_body(i, st):
                acc, b8 = st
                cb = (RBUCKETS // L - 1) - i
                h = hist_v[pl.ds(cb * L, L)]
                hr = lax.rev(h, (0,))
                cumr = plsc.cumsum(hr)
                tot = cumr[15]
                cross = (acc + cumr) >= K
                take = jnp.logical_and(acc + tot >= K, b8 < 0)
                f = _scalar(plsc.all_reduce_ffs(cross))
                b8 = jnp.where(take, cb * L + (15 - f), b8)
                return (acc + tot, b8)

            _, b8 = lax.fori_loop(
                0, RBUCKETS // L, scan_body,
                (jnp.int32(0), jnp.int32(-1)))

            def refine_sub(_):
                def sub_pass(cc, st):
                    off2, nabove = st
                    v = cand_v[pl.ds(cc * L, L)]
                    key = _keys(v)
                    valid = (cc * L + iota16) < nc0
                    bkt = jnp.minimum(
                        (key - jnp.int32(T0KEY)) >> RBITS,
                        jnp.int32(RBUCKETS - 1))
                    m = jnp.logical_and(bkt == b8, valid)
                    gt = jnp.logical_and(bkt > b8, valid)
                    plsc.store_compressed(
                        buf_v.at[pl.ds(off2, L)], v, mask=m)
                    return (
                        off2 + _scalar(
                            plsc.all_reduce_population_count(m)),
                        nabove + _scalar(
                            plsc.all_reduce_population_count(gt)))

                nsub, n_above = lax.fori_loop(
                    0, ncc, sub_pass, (jnp.int32(0), jnp.int32(0)))
                nsc = (nsub + (L - 1)) // L
                t1 = jnp.int32(T0KEY) + (b8 << RBITS)
                ktarget = K - n_above
                tkey = bit_search(buf_v, t1, RBITS, nsc, nsub, ktarget)
                c_gt = n_above + count_ge_ref(
                    buf_v, tkey + 1, nsc, nsub)
                c_ge = n_above + count_ge_ref(buf_v, tkey, nsc, nsub)
                return tkey, c_gt, c_ge

            def refine_full(_):
                tkey = bit_search(
                    cand_v, jnp.int32(T0KEY), 30, ncc, nc0, K)
                return (tkey,
                        count_ge_ref(cand_v, tkey + 1, ncc, nc0),
                        count_ge_ref(cand_v, tkey, ncc, nc0))

            tkey, c_gt, c_ge = lax.cond(
                b8 < jnp.int32(RBUCKETS - 1), refine_sub, refine_full, 0)
            emit_selection(cand_v, nc0, ncc, tkey, c_gt, c_ge)
            return 0

        def path_exact(_):
            c_pos = count_ge_ref(buf_v, jnp.int32(0), CHUNKS, N)

            def pos_case(_c):
                return bit_search(buf_v, jnp.int32(0), 31, CHUNKS, N, K)

            def neg_case(_c):
                return bit_search(buf_v, jnp.int32(IMIN), 31, CHUNKS, N, K)

            tkey = lax.cond(c_pos >= K, pos_case, neg_case, 0)
            c_gt = count_ge_ref(buf_v, tkey + 1, CHUNKS, N)
            c_ge = count_ge_ref(buf_v, tkey, CHUNKS, N)
            emit_selection(buf_v, N, CHUNKS, tkey, c_gt, c_ge)
            return 0

        use_fast = jnp.logical_and(nc0 >= K, jnp.logical_not(ovf))
        lax.cond(use_fast, path_fast, path_exact, 0)
        pltpu.sync_copy(outrow_v.at[pl.ds(0, K)], out_hbm.at[row])

    row0 = wid * ROWS_PER_W
    pltpu.sync_copy(x_hbm.at[row0], rowa_v)
    pltpu.sync_copy(x_hbm.at[row0 + 1], rowb_v)
    for p in range(ROWS_PER_W // 2):
        ra = row0 + 2 * p
        rb = ra + 1

        @plsc.parallel_loop(0, CHUNKS, step=1, unroll=4,
                            carry=(jnp.int32(0), jnp.int32(0)))
        def offs(c, st):
            offa, offb = st
            va = rowa_v[pl.ds(c * L, L)]
            ma = va >= jnp.float32(T0)
            plsc.store_compressed(
                canda_v.at[pl.ds(jnp.minimum(offa, CAP - L), L)], va,
                mask=ma)
            vb = rowb_v[pl.ds(c * L, L)]
            mb = vb >= jnp.float32(T0)
            plsc.store_compressed(
                candb_v.at[pl.ds(jnp.minimum(offb, CAP - L), L)], vb,
                mask=mb)
            return (
                offa + _scalar(plsc.all_reduce_population_count(ma)),
                offb + _scalar(plsc.all_reduce_population_count(mb)))

        nca, ncb = offs
        do_tail(ra, rowa_v, canda_v, nca, nca > CAP - L)
        ha = None
        if p + 1 < ROWS_PER_W // 2:
            ha = pltpu.async_copy(x_hbm.at[ra + 2], rowa_v, sema)
        do_tail(rb, rowb_v, candb_v, ncb, ncb > CAP - L)
        if ha is not None:
            pltpu.async_copy(x_hbm.at[rb + 2], rowb_v, semb).wait()
            ha.wait()


_mesh = plsc.VectorSubcoreMesh(
    core_axis_name="c", subcore_axis_name="s", num_cores=NC, num_subcores=NS)

_kmax = pl.kernel(
    _body,
    out_type=jax.ShapeDtypeStruct((R, K), jnp.float32),
    mesh=_mesh,
    scratch_types=[
        pltpu.VMEM((N,), jnp.float32),
        pltpu.VMEM((N,), jnp.float32),
        pltpu.VMEM((RBUCKETS,), jnp.int32),
        pltpu.VMEM((CAP,), jnp.float32),
        pltpu.VMEM((CAP,), jnp.float32),
        pltpu.VMEM((K + L,), jnp.float32),
        pltpu.SemaphoreType.DMA,
        pltpu.SemaphoreType.DMA,
    ],
    compiler_params=pltpu.CompilerParams(needs_layout_passes=False),
)


@jax.jit
def kernel(x):
    return _kmax(x)
